# Initial kernel scaffold; baseline (speedup 1.0000x reference)
import jax
import jax.numpy as jnp
from jax.experimental import pallas as pl


def kernel(curr_coords, curr_feats, skip_coords, skip_feats, upsampling_idxs, W_proj, b_proj, g_proj, be_proj, W_skip, b_skip, g_skip, be_skip):
    raise NotImplementedError("write your pallas kernel here")


# speedup vs baseline: 15.9836x; 15.9836x over previous
import functools

import jax
import jax.numpy as jnp
from jax import lax
from jax.experimental import pallas as pl
from jax.experimental.pallas import tpu as pltpu
from jax.experimental.pallas import tpu_sc as plsc

_EPS = 1e-5

_NUM_CORES = 2
_NUM_SUBCORES = 16
_NW = _NUM_CORES * _NUM_SUBCORES
_CHUNK = 128


def _gather_max_sc(table, idx2d, B, N, K, C):
    BN = B * N
    ppw = BN // _NW
    wpb = N // ppw
    ch = ppw // _CHUNK
    rows_per_k = ppw // 128

    mesh = plsc.VectorSubcoreMesh(
        core_axis_name="c", subcore_axis_name="s",
        num_cores=_NUM_CORES, num_subcores=_NUM_SUBCORES)

    @functools.partial(
        pl.kernel,
        mesh=mesh,
        compiler_params=pltpu.CompilerParams(use_tc_tiling_on_sc=False),
        out_type=jax.ShapeDtypeStruct((BN, C), jnp.float32),
        scratch_types=[
            pltpu.VMEM((K * rows_per_k, 128), jnp.int32),
            pltpu.VMEM((_CHUNK, C), jnp.float32),
            pltpu.VMEM((_CHUNK, C), jnp.float32),
            pltpu.VMEM((_CHUNK, C), jnp.float32),
            pltpu.SemaphoreType.DMA,
            pltpu.SemaphoreType.DMA,
            pltpu.SemaphoreType.DMA,
        ],
    )
    def sc_kernel(table_h, idx_h, out_h, idx_all, r0, r1, r2, s0, s1, s2):
        wid = lax.axis_index("s") * _NUM_CORES + lax.axis_index("c")
        b = wid // wpb
        n0 = (wid % wpb) * ppw
        for k in range(K):
            src_row = pl.multiple_of((b * K * N + k * N + n0) // 128, 8)
            pltpu.sync_copy(idx_h.at[pl.ds(src_row, rows_per_k), :],
                            idx_all.at[pl.ds(k * rows_per_k, rows_per_k), :])

        def chunk(c, carry):
            cp0 = pltpu.async_copy(table_h.at[idx_all.at[c]], r0, s0)
            cp1 = pltpu.async_copy(table_h.at[idx_all.at[rows_per_k + c]], r1, s1)
            cp2 = pltpu.async_copy(table_h.at[idx_all.at[2 * rows_per_k + c]], r2, s2)
            cp0.wait()
            cp1.wait()
            cp2.wait()

            def pbody(p, pc):
                for cc in range(C // 16):
                    sl = pl.ds(cc * 16, 16)
                    v = jnp.maximum(jnp.maximum(r0[p, sl], r1[p, sl]), r2[p, sl])
                    r0[p, sl] = v
                return pc
            lax.fori_loop(0, _CHUNK, pbody, 0, unroll=2)

            obase = pl.multiple_of(b * N + n0 + c * _CHUNK, 8)
            pltpu.sync_copy(r0, out_h.at[pl.ds(obase, _CHUNK), :])
            return carry
        lax.fori_loop(0, ch, chunk, 0)

    return sc_kernel(table, idx2d)


def _stats_tc(inter, skipf, wp, bp, ws, bs):
    BN, _ = inter.shape
    Co = wp.shape[1]
    R = 2048
    nb = BN // R

    def body(x_ref, s_ref, wp_ref, bp_ref, ws_ref, bs_ref, o_ref):
        i = pl.program_id(0)
        yi = jnp.dot(x_ref[...], wp_ref[...],
                     preferred_element_type=jnp.float32) + bp_ref[...]
        ys = jnp.dot(s_ref[...], ws_ref[...],
                     preferred_element_type=jnp.float32) + bs_ref[...]
        part = jnp.concatenate([
            jnp.sum(yi, axis=0, keepdims=True),
            jnp.sum(yi * yi, axis=0, keepdims=True),
            jnp.sum(ys, axis=0, keepdims=True),
            jnp.sum(ys * ys, axis=0, keepdims=True),
        ], axis=0)

        @pl.when(i == 0)
        def _():
            o_ref[...] = jnp.zeros_like(o_ref)

        o_ref[...] += part

    return pl.pallas_call(
        body,
        grid=(nb,),
        in_specs=[
            pl.BlockSpec((R, inter.shape[1]), lambda i: (i, 0)),
            pl.BlockSpec((R, skipf.shape[1]), lambda i: (i, 0)),
            pl.BlockSpec(wp.shape, lambda i: (0, 0)),
            pl.BlockSpec(bp.shape, lambda i: (0, 0)),
            pl.BlockSpec(ws.shape, lambda i: (0, 0)),
            pl.BlockSpec(bs.shape, lambda i: (0, 0)),
        ],
        out_specs=pl.BlockSpec((4, Co), lambda i: (0, 0)),
        out_shape=jax.ShapeDtypeStruct((4, Co), jnp.float32),
    )(inter, skipf, wp, bp, ws, bs)


def _apply_tc(inter, skipf, coords, stats, wp, bp, ws, bs, gp, bep, gs, bes):
    BN, Ci = inter.shape
    Cs = skipf.shape[1]
    Co = wp.shape[1]
    Cc = coords.shape[1]
    R = 1024
    nb = BN // R
    cnt = float(BN)

    def body(st_ref, x_ref, s_ref, c_ref, wp_ref, bp_ref, ws_ref, bs_ref,
             gp_ref, bep_ref, gs_ref, bes_ref, o_ref):
        st = st_ref[...]
        m_i = st[0:1, :] * (1.0 / cnt)
        v_i = st[1:2, :] * (1.0 / cnt) - m_i * m_i
        m_s = st[2:3, :] * (1.0 / cnt)
        v_s = st[3:4, :] * (1.0 / cnt) - m_s * m_s
        sc_i = gp_ref[...] * lax.rsqrt(v_i + _EPS)
        sh_i = bep_ref[...] - m_i * sc_i
        sc_s = gs_ref[...] * lax.rsqrt(v_s + _EPS)
        sh_s = bes_ref[...] - m_s * sc_s

        yi = jnp.dot(x_ref[...], wp_ref[...],
                     preferred_element_type=jnp.float32) + bp_ref[...]
        ys = jnp.dot(s_ref[...], ws_ref[...],
                     preferred_element_type=jnp.float32) + bs_ref[...]
        zi = jnp.maximum(yi * sc_i + sh_i, 0.0)
        zs = jnp.maximum(ys * sc_s + sh_s, 0.0)
        o_ref[...] = jnp.concatenate([c_ref[...], zs, zi], axis=1)

    return pl.pallas_call(
        body,
        grid=(nb,),
        in_specs=[
            pl.BlockSpec((4, Co), lambda i: (0, 0)),
            pl.BlockSpec((R, Ci), lambda i: (i, 0)),
            pl.BlockSpec((R, Cs), lambda i: (i, 0)),
            pl.BlockSpec((R, Cc), lambda i: (i, 0)),
            pl.BlockSpec(wp.shape, lambda i: (0, 0)),
            pl.BlockSpec(bp.shape, lambda i: (0, 0)),
            pl.BlockSpec(ws.shape, lambda i: (0, 0)),
            pl.BlockSpec(bs.shape, lambda i: (0, 0)),
            pl.BlockSpec(gp.shape, lambda i: (0, 0)),
            pl.BlockSpec(bep.shape, lambda i: (0, 0)),
            pl.BlockSpec(gs.shape, lambda i: (0, 0)),
            pl.BlockSpec(bes.shape, lambda i: (0, 0)),
        ],
        out_specs=pl.BlockSpec((R, Cc + 2 * Co), lambda i: (i, 0)),
        out_shape=jax.ShapeDtypeStruct((BN, Cc + 2 * Co), jnp.float32),
    )(stats, inter, skipf, coords, wp, bp, ws, bs, gp, bep, gs, bes)


def kernel(curr_coords, curr_feats, skip_coords, skip_feats, upsampling_idxs,
           W_proj, b_proj, g_proj, be_proj, W_skip, b_skip, g_skip, be_skip):
    B, M, C = curr_feats.shape
    _, N, K = upsampling_idxs.shape
    BN = B * N
    Co = W_proj.shape[1]

    offs = (jnp.arange(B, dtype=jnp.int32) * M)[:, None, None]
    idx2d = (upsampling_idxs + offs).transpose(0, 2, 1).reshape(-1, 128)
    table = curr_feats.reshape(B * M, C)

    inter = _gather_max_sc(table, idx2d, B, N, K, C)

    skipf = skip_feats.reshape(BN, -1)
    coords = skip_coords.reshape(BN, -1)
    r1 = lambda a: a.reshape(1, -1)
    stats = _stats_tc(inter, skipf, W_proj, r1(b_proj), W_skip, r1(b_skip))
    out = _apply_tc(inter, skipf, coords, stats,
                    W_proj, r1(b_proj), W_skip, r1(b_skip),
                    r1(g_proj), r1(be_proj), r1(g_skip), r1(be_skip))
    return out.reshape(B, N, 3 + 2 * Co)


# speedup vs baseline: 17.6761x; 1.1059x over previous
import functools

import jax
import jax.numpy as jnp
from jax import lax
from jax.experimental import pallas as pl
from jax.experimental.pallas import tpu as pltpu
from jax.experimental.pallas import tpu_sc as plsc

_EPS = 1e-5

_NUM_CORES = 2
_NUM_SUBCORES = 16
_NW = _NUM_CORES * _NUM_SUBCORES
_CHUNK = 128


def _gather_max_sc(table, idx2d, B, M, N, K, C):
    BN = B * N
    ppw = BN // _NW
    wpb = N // ppw
    ch = ppw // _CHUNK
    idx_rows = ppw * K // 128
    rpc = _CHUNK * K // 128

    mesh = plsc.VectorSubcoreMesh(
        core_axis_name="c", subcore_axis_name="s",
        num_cores=_NUM_CORES, num_subcores=_NUM_SUBCORES)

    @functools.partial(
        pl.kernel,
        mesh=mesh,
        compiler_params=pltpu.CompilerParams(use_tc_tiling_on_sc=False),
        out_type=jax.ShapeDtypeStruct((BN, C), jnp.float32),
        scratch_types=[
            pltpu.VMEM((idx_rows, 128), jnp.int32),
            pltpu.VMEM((_CHUNK * K, C), jnp.float32),
            pltpu.VMEM((_CHUNK * K, C), jnp.float32),
            pltpu.VMEM((_CHUNK, C), jnp.float32),
            pltpu.VMEM((_CHUNK, C), jnp.float32),
            pltpu.SemaphoreType.DMA,
            pltpu.SemaphoreType.DMA,
        ],
    )
    def sc_kernel(table_h, idx_h, out_h, idx_all, ra, rb, oa, ob, sa, sb):
        wid = lax.axis_index("s") * _NUM_CORES + lax.axis_index("c")
        b = wid // wpb
        n0 = (wid % wpb) * ppw
        tb = table_h.at[pl.ds(pl.multiple_of(b * M, 8), M), :]
        rbufs = (ra, rb)
        obufs = (oa, ob)
        sems = (sa, sb)

        src_row = pl.multiple_of((b * N + n0) * K // 128, 8)
        pltpu.sync_copy(idx_h.at[pl.ds(src_row, idx_rows), :], idx_all)

        def fire(c, s):
            for j in range(rpc):
                pltpu.async_copy(
                    tb.at[idx_all.at[c * rpc + j]],
                    rbufs[s].at[pl.ds(j * 128, 128), :], sems[s])

        def drain(s):
            for j in range(rpc):
                pltpu.make_async_copy(
                    tb.at[idx_all.at[j]],
                    rbufs[s].at[pl.ds(j * 128, 128), :], sems[s]).wait()

        fire(0, 0)

        def outer(i, carry):
            for s in range(2):
                c = 2 * i + s
                drain(s)

                @pl.when(c + 1 < ch)
                def _():
                    fire(c + 1, 1 - s)

                rbuf, obuf = rbufs[s], obufs[s]

                def pbody(p, pc):
                    for cc in range(C // 16):
                        sl = pl.ds(cc * 16, 16)
                        v = jnp.maximum(
                            jnp.maximum(rbuf[3 * p, sl], rbuf[3 * p + 1, sl]),
                            rbuf[3 * p + 2, sl])
                        obuf[p, sl] = v
                    return pc
                lax.fori_loop(0, _CHUNK, pbody, 0, unroll=2)

                obase = pl.multiple_of(b * N + n0 + c * _CHUNK, 8)
                pltpu.sync_copy(obuf, out_h.at[pl.ds(obase, _CHUNK), :])
            return carry
        lax.fori_loop(0, ch // 2, outer, 0)

    return sc_kernel(table, idx2d)


def _stats_tc(inter, skipf, wp, bp, ws, bs):
    BN, _ = inter.shape
    Co = wp.shape[1]
    R = 4096
    nb = BN // R

    def body(x_ref, s_ref, wp_ref, bp_ref, ws_ref, bs_ref, o_ref):
        i = pl.program_id(0)
        yi = jnp.dot(x_ref[...], wp_ref[...],
                     preferred_element_type=jnp.float32) + bp_ref[...]
        ys = jnp.dot(s_ref[...], ws_ref[...],
                     preferred_element_type=jnp.float32) + bs_ref[...]
        part = jnp.concatenate([
            jnp.sum(yi, axis=0, keepdims=True),
            jnp.sum(yi * yi, axis=0, keepdims=True),
            jnp.sum(ys, axis=0, keepdims=True),
            jnp.sum(ys * ys, axis=0, keepdims=True),
        ], axis=0)

        @pl.when(i == 0)
        def _():
            o_ref[...] = jnp.zeros_like(o_ref)

        o_ref[...] += part

    return pl.pallas_call(
        body,
        grid=(nb,),
        in_specs=[
            pl.BlockSpec((R, inter.shape[1]), lambda i: (i, 0)),
            pl.BlockSpec((R, skipf.shape[1]), lambda i: (i, 0)),
            pl.BlockSpec(wp.shape, lambda i: (0, 0)),
            pl.BlockSpec(bp.shape, lambda i: (0, 0)),
            pl.BlockSpec(ws.shape, lambda i: (0, 0)),
            pl.BlockSpec(bs.shape, lambda i: (0, 0)),
        ],
        out_specs=pl.BlockSpec((4, Co), lambda i: (0, 0)),
        out_shape=jax.ShapeDtypeStruct((4, Co), jnp.float32),
    )(inter, skipf, wp, bp, ws, bs)


def _apply_tc(inter, skipf, coords, stats, wp, bp, ws, bs, gp, bep, gs, bes):
    BN, Ci = inter.shape
    Cs = skipf.shape[1]
    Co = wp.shape[1]
    Cc = coords.shape[1]
    R = 4096
    nb = BN // R
    cnt = float(BN)

    def body(st_ref, x_ref, s_ref, c_ref, wp_ref, bp_ref, ws_ref, bs_ref,
             gp_ref, bep_ref, gs_ref, bes_ref, o_ref):
        st = st_ref[...]
        m_i = st[0:1, :] * (1.0 / cnt)
        v_i = st[1:2, :] * (1.0 / cnt) - m_i * m_i
        m_s = st[2:3, :] * (1.0 / cnt)
        v_s = st[3:4, :] * (1.0 / cnt) - m_s * m_s
        sc_i = gp_ref[...] * lax.rsqrt(v_i + _EPS)
        sh_i = bep_ref[...] - m_i * sc_i
        sc_s = gs_ref[...] * lax.rsqrt(v_s + _EPS)
        sh_s = bes_ref[...] - m_s * sc_s

        yi = jnp.dot(x_ref[...], wp_ref[...],
                     preferred_element_type=jnp.float32) + bp_ref[...]
        ys = jnp.dot(s_ref[...], ws_ref[...],
                     preferred_element_type=jnp.float32) + bs_ref[...]
        zi = jnp.maximum(yi * sc_i + sh_i, 0.0)
        zs = jnp.maximum(ys * sc_s + sh_s, 0.0)
        o_ref[...] = jnp.concatenate([c_ref[...], zs, zi], axis=1)

    return pl.pallas_call(
        body,
        grid=(nb,),
        in_specs=[
            pl.BlockSpec((4, Co), lambda i: (0, 0)),
            pl.BlockSpec((R, Ci), lambda i: (i, 0)),
            pl.BlockSpec((R, Cs), lambda i: (i, 0)),
            pl.BlockSpec((R, Cc), lambda i: (i, 0)),
            pl.BlockSpec(wp.shape, lambda i: (0, 0)),
            pl.BlockSpec(bp.shape, lambda i: (0, 0)),
            pl.BlockSpec(ws.shape, lambda i: (0, 0)),
            pl.BlockSpec(bs.shape, lambda i: (0, 0)),
            pl.BlockSpec(gp.shape, lambda i: (0, 0)),
            pl.BlockSpec(bep.shape, lambda i: (0, 0)),
            pl.BlockSpec(gs.shape, lambda i: (0, 0)),
            pl.BlockSpec(bes.shape, lambda i: (0, 0)),
        ],
        out_specs=pl.BlockSpec((R, Cc + 2 * Co), lambda i: (i, 0)),
        out_shape=jax.ShapeDtypeStruct((BN, Cc + 2 * Co), jnp.float32),
    )(stats, inter, skipf, coords, wp, bp, ws, bs, gp, bep, gs, bes)


def kernel(curr_coords, curr_feats, skip_coords, skip_feats, upsampling_idxs,
           W_proj, b_proj, g_proj, be_proj, W_skip, b_skip, g_skip, be_skip):
    B, M, C = curr_feats.shape
    _, N, K = upsampling_idxs.shape
    BN = B * N
    Co = W_proj.shape[1]

    idx2d = upsampling_idxs.reshape(-1, 128)
    table = curr_feats.reshape(B * M, C)

    inter = _gather_max_sc(table, idx2d, B, M, N, K, C)

    skipf = skip_feats.reshape(BN, -1)
    coords = skip_coords.reshape(BN, -1)
    r1 = lambda a: a.reshape(1, -1)
    stats = _stats_tc(inter, skipf, W_proj, r1(b_proj), W_skip, r1(b_skip))
    out = _apply_tc(inter, skipf, coords, stats,
                    W_proj, r1(b_proj), W_skip, r1(b_skip),
                    r1(g_proj), r1(be_proj), r1(g_skip), r1(be_skip))
    return out.reshape(B, N, 3 + 2 * Co)


# speedup vs baseline: 24.6371x; 1.3938x over previous
import functools

import jax
import jax.numpy as jnp
from jax import lax
from jax.experimental import pallas as pl
from jax.experimental.pallas import tpu as pltpu
from jax.experimental.pallas import tpu_sc as plsc

_EPS = 1e-5

_NUM_CORES = 2
_NUM_SUBCORES = 16
_NW = _NUM_CORES * _NUM_SUBCORES
_CHUNK = 128


def _gather_max_sc(table, idx3, B, M, N, K, C):
    BN = B * N
    ppw = BN // _NW
    wpb = N // ppw
    ch = ppw // _CHUNK

    mesh = plsc.VectorSubcoreMesh(
        core_axis_name="c", subcore_axis_name="s",
        num_cores=_NUM_CORES, num_subcores=_NUM_SUBCORES)

    @functools.partial(
        pl.kernel,
        mesh=mesh,
        compiler_params=pltpu.CompilerParams(use_tc_tiling_on_sc=False,
                                             needs_layout_passes=False),
        out_type=jax.ShapeDtypeStruct((BN // 2, 2 * C), jnp.float32),
        scratch_types=[
            pltpu.VMEM((K * ch, 128), jnp.int32),
            pltpu.VMEM((K * _CHUNK, C), jnp.float32),
            pltpu.VMEM((K * _CHUNK, C), jnp.float32),
            pltpu.VMEM((C, _CHUNK), jnp.float32),
            pltpu.VMEM((C, _CHUNK), jnp.float32),
            pltpu.SemaphoreType.DMA,
            pltpu.SemaphoreType.DMA,
        ],
    )
    def sc_kernel(table_h, idx_h, out_h, idx_all, ra, rb, oa, ob, sa, sb):
        wid = lax.axis_index("s") * _NUM_CORES + lax.axis_index("c")
        b = wid // wpb
        n0 = (wid % wpb) * ppw
        g0 = pl.multiple_of(n0 // 128, 8)
        tb = table_h.at[pl.ds(pl.multiple_of(b * M, 8), M), :]
        rbufs = (ra, rb)
        obufs = (oa, ob)
        sems = (sa, sb)
        lane = lax.iota(jnp.int32, 16)
        rowidx = [cc * 16 + lane for cc in range(C // 16)]

        for k in range(K):
            pltpu.sync_copy(
                idx_h.at[k, pl.ds(g0, ch), pl.ds(pl.multiple_of(b * 128, 8), 128)],
                idx_all.at[pl.ds(k * ch, ch), :])

        def fire(c, s):
            for k in range(K):
                pltpu.async_copy(
                    tb.at[idx_all.at[k * ch + c]],
                    rbufs[s].at[pl.ds(k * _CHUNK, _CHUNK), :], sems[s])

        def drain(s):
            for k in range(K):
                pltpu.make_async_copy(
                    tb.at[idx_all.at[k * ch]],
                    rbufs[s].at[pl.ds(k * _CHUNK, _CHUNK), :], sems[s]).wait()

        fire(0, 0)

        def outer(i, carry):
            for s in range(2):
                c = 2 * i + s
                drain(s)

                @pl.when(c + 1 < ch)
                def _():
                    fire(c + 1, 1 - s)

                rbuf, obuf = rbufs[s], obufs[s]

                def pbody(p, pc):
                    pcol = jnp.full((16,), p, jnp.int32)
                    for cc in range(C // 16):
                        sl = pl.ds(cc * 16, 16)
                        v = jnp.maximum(
                            jnp.maximum(rbuf[p, sl], rbuf[_CHUNK + p, sl]),
                            rbuf[2 * _CHUNK + p, sl])
                        plsc.store_scatter(obuf, [rowidx[cc], pcol], v)
                    return pc
                lax.fori_loop(0, _CHUNK, pbody, 0, unroll=2)

                obase = pl.multiple_of(((g0 + c) * B + b) * C, 8)
                pltpu.sync_copy(obuf, out_h.at[pl.ds(obase, C), :])
            return carry
        lax.fori_loop(0, ch // 2, outer, 0)

    return sc_kernel(table, idx3)


_GBLK = 8


def _stats_tc(inter, skipt, wpt, bpt, wst, bst, B, N, C):
    Co = wpt.shape[0]
    Cs = skipt.shape[1]
    NG = N // 128
    nb = NG // _GBLK

    def body(x_ref, s_ref, wpt_ref, bpt_ref, wst_ref, bst_ref, oi_ref, os_ref):
        first = pl.program_id(0) == 0
        x4 = x_ref[...].reshape(_GBLK, B, C, 128)
        s1i = jnp.zeros((Co, 1), jnp.float32)
        s2i = jnp.zeros((Co, 1), jnp.float32)
        for gi in range(_GBLK):
            for b in range(B):
                yit = jnp.dot(wpt_ref[...], x4[gi, b],
                              preferred_element_type=jnp.float32) + bpt_ref[...]
                s1i += jnp.sum(yit, axis=1, keepdims=True)
                s2i += jnp.sum(yit * yit, axis=1, keepdims=True)
        s1s = jnp.zeros((Cs, 1), jnp.float32)
        s2s = jnp.zeros((Cs, 1), jnp.float32)
        for b in range(B):
            yst = jnp.dot(wst_ref[...], s_ref[b],
                          preferred_element_type=jnp.float32) + bst_ref[...]
            s1s += jnp.sum(yst, axis=1, keepdims=True)
            s2s += jnp.sum(yst * yst, axis=1, keepdims=True)

        @pl.when(first)
        def _():
            oi_ref[...] = jnp.zeros_like(oi_ref)
            os_ref[...] = jnp.zeros_like(os_ref)

        oi_ref[...] += jnp.concatenate([s1i, s2i], axis=1)
        os_ref[...] += jnp.concatenate([s1s, s2s], axis=1)

    return pl.pallas_call(
        body,
        grid=(nb,),
        in_specs=[
            pl.BlockSpec((_GBLK * B * C, 128), lambda j: (j, 0)),
            pl.BlockSpec((B, Cs, _GBLK * 128), lambda j: (0, 0, j)),
            pl.BlockSpec(wpt.shape, lambda j: (0, 0)),
            pl.BlockSpec(bpt.shape, lambda j: (0, 0)),
            pl.BlockSpec(wst.shape, lambda j: (0, 0)),
            pl.BlockSpec(bst.shape, lambda j: (0, 0)),
        ],
        out_specs=[
            pl.BlockSpec((Co, 2), lambda j: (0, 0)),
            pl.BlockSpec((Cs, 2), lambda j: (0, 0)),
        ],
        out_shape=[
            jax.ShapeDtypeStruct((Co, 2), jnp.float32),
            jax.ShapeDtypeStruct((Cs, 2), jnp.float32),
        ],
    )(inter, skipt, wpt, bpt, wst, bst)


def _apply_tc(inter, skipt, coords3, sti, sts, wpt, bpt, wst, bst,
              gp, bep, gs, bes, B, N, C):
    Co = wpt.shape[0]
    Cs = skipt.shape[1]
    NG = N // 128
    nb = NG // _GBLK
    cnt = float(B * N)
    GB = _GBLK * B

    def body(sti_ref, sts_ref, x_ref, s_ref, c_ref, wpt_ref, bpt_ref,
             wst_ref, bst_ref, gp_ref, bep_ref, gs_ref, bes_ref, o_ref):
        sti_v = sti_ref[...]
        m_i = sti_v[:, 0:1] * (1.0 / cnt)
        v_i = sti_v[:, 1:2] * (1.0 / cnt) - m_i * m_i
        sc_i = gp_ref[...] * lax.rsqrt(v_i + _EPS)
        sh_i = bep_ref[...] - m_i * sc_i
        sts_v = sts_ref[...]
        m_s = sts_v[:, 0:1] * (1.0 / cnt)
        v_s = sts_v[:, 1:2] * (1.0 / cnt) - m_s * m_s
        sc_s = gs_ref[...] * lax.rsqrt(v_s + _EPS)
        sh_s = bes_ref[...] - m_s * sc_s

        o_ref[0:3, :, :] = c_ref[...]

        for b in range(B):
            yst = jnp.dot(wst_ref[...], s_ref[b],
                          preferred_element_type=jnp.float32) + bst_ref[...]
            zst = jnp.maximum(yst * sc_s + sh_s, 0.0)
            for gi in range(_GBLK):
                o_ref[3:3 + Cs, pl.ds(gi * B + b, 1), :] = (
                    zst[:, gi * 128:(gi + 1) * 128])[:, None, :]

        x4 = x_ref[...].reshape(_GBLK, B, C, 128)
        for gi in range(_GBLK):
            for b in range(B):
                yit = jnp.dot(wpt_ref[...], x4[gi, b],
                              preferred_element_type=jnp.float32) + bpt_ref[...]
                zit = jnp.maximum(yit * sc_i + sh_i, 0.0)
                o_ref[3 + Cs:3 + Cs + Co, pl.ds(gi * B + b, 1), :] = zit[:, None, :]

    return pl.pallas_call(
        body,
        grid=(nb,),
        in_specs=[
            pl.BlockSpec((Co, 2), lambda j: (0, 0)),
            pl.BlockSpec((Cs, 2), lambda j: (0, 0)),
            pl.BlockSpec((_GBLK * B * C, 128), lambda j: (j, 0)),
            pl.BlockSpec((B, Cs, _GBLK * 128), lambda j: (0, 0, j)),
            pl.BlockSpec((3, GB, 128), lambda j: (0, j, 0)),
            pl.BlockSpec(wpt.shape, lambda j: (0, 0)),
            pl.BlockSpec(bpt.shape, lambda j: (0, 0)),
            pl.BlockSpec(wst.shape, lambda j: (0, 0)),
            pl.BlockSpec(bst.shape, lambda j: (0, 0)),
            pl.BlockSpec(gp.shape, lambda j: (0, 0)),
            pl.BlockSpec(bep.shape, lambda j: (0, 0)),
            pl.BlockSpec(gs.shape, lambda j: (0, 0)),
            pl.BlockSpec(bes.shape, lambda j: (0, 0)),
        ],
        out_specs=pl.BlockSpec((3 + Cs + Co, GB, 128), lambda j: (0, j, 0)),
        out_shape=jax.ShapeDtypeStruct((3 + Cs + Co, NG * B, 128), jnp.float32),
    )(sti, sts, inter, skipt, coords3, wpt, bpt, wst, bst, gp, bep, gs, bes)


def kernel(curr_coords, curr_feats, skip_coords, skip_feats, upsampling_idxs,
           W_proj, b_proj, g_proj, be_proj, W_skip, b_skip, g_skip, be_skip):
    B, M, C = curr_feats.shape
    _, N, K = upsampling_idxs.shape
    Co = W_proj.shape[1]

    idx3 = (upsampling_idxs.reshape(B, N // 128, 128, K)
            .transpose(3, 1, 0, 2).reshape(K, N // 128, B * 128))
    table = curr_feats.reshape(B * M, C)
    skipt = jnp.transpose(skip_feats, (0, 2, 1))
    coords3 = (skip_coords.reshape(B, N // 128, 128, 3)
               .transpose(3, 1, 0, 2).reshape(3, (N // 128) * B, 128))

    inter = _gather_max_sc(table, idx3, B, M, N, K, C)

    wpt = W_proj.T
    wst = W_skip.T
    col = lambda a: a.reshape(-1, 1)
    sti, sts = _stats_tc(inter, skipt, wpt, col(b_proj), wst, col(b_skip),
                         B, N, C)
    out3 = _apply_tc(inter, skipt, coords3, sti, sts,
                     wpt, col(b_proj), wst, col(b_skip),
                     col(g_proj), col(be_proj), col(g_skip), col(be_skip),
                     B, N, C)
    return (out3.reshape(3 + 2 * Co, N // 128, B, 128)
            .transpose(2, 1, 3, 0).reshape(B, N, 3 + 2 * Co))


# speedup vs baseline: 35.5413x; 1.4426x over previous
import functools

import jax
import jax.numpy as jnp
from jax import lax
from jax.experimental import pallas as pl
from jax.experimental.pallas import tpu as pltpu
from jax.experimental.pallas import tpu_sc as plsc

_EPS = 1e-5

_NUM_CORES = 2
_NUM_SUBCORES = 16
_NW = _NUM_CORES * _NUM_SUBCORES
_CHUNK = 128


def _gather_max_sc(table, idx3, B, M, N, K, C):
    BN = B * N
    ppw = BN // _NW
    wpb = N // ppw
    ch = ppw // _CHUNK

    mesh = plsc.VectorSubcoreMesh(
        core_axis_name="c", subcore_axis_name="s",
        num_cores=_NUM_CORES, num_subcores=_NUM_SUBCORES)

    @functools.partial(
        pl.kernel,
        mesh=mesh,
        compiler_params=pltpu.CompilerParams(use_tc_tiling_on_sc=False,
                                             needs_layout_passes=False),
        out_type=jax.ShapeDtypeStruct((BN // 2, 2 * C), jnp.float32),
        scratch_types=[
            pltpu.VMEM((K * ch, 128), jnp.int32),
            pltpu.VMEM((K * _CHUNK, C), jnp.float32),
            pltpu.VMEM((K * _CHUNK, C), jnp.float32),
            pltpu.VMEM((C, _CHUNK + 1), jnp.float32),
            pltpu.VMEM((C, _CHUNK + 1), jnp.float32),
            pltpu.SemaphoreType.DMA,
            pltpu.SemaphoreType.DMA,
        ],
    )
    def sc_kernel(table_h, idx_h, out_h, idx_all, ra, rb, oa, ob, sa, sb):
        wid = lax.axis_index("s") * _NUM_CORES + lax.axis_index("c")
        b = wid // wpb
        n0 = (wid % wpb) * ppw
        g0 = pl.multiple_of(n0 // 128, 8)
        tb = table_h.at[pl.ds(pl.multiple_of(b * M, 8), M), :]
        rbufs = (ra, rb)
        obufs = (oa, ob)
        sems = (sa, sb)
        lane = lax.iota(jnp.int32, 16)
        rowidx = [cc * 16 + lane for cc in range(C // 16)]

        for k in range(K):
            pltpu.sync_copy(
                idx_h.at[k, pl.ds(g0, ch), pl.ds(pl.multiple_of(b * 128, 8), 128)],
                idx_all.at[pl.ds(k * ch, ch), :])

        def fire(c, s):
            for k in range(K):
                pltpu.async_copy(
                    tb.at[idx_all.at[k * ch + c]],
                    rbufs[s].at[pl.ds(k * _CHUNK, _CHUNK), :], sems[s])

        def drain(s):
            for k in range(K):
                pltpu.make_async_copy(
                    tb.at[idx_all.at[k * ch]],
                    rbufs[s].at[pl.ds(k * _CHUNK, _CHUNK), :], sems[s]).wait()

        fire(0, 0)

        def outer(i, carry):
            for s in range(2):
                c = 2 * i + s
                drain(s)

                @pl.when(c + 1 < ch)
                def _():
                    fire(c + 1, 1 - s)

                rbuf, obuf = rbufs[s], obufs[s]

                def pbody(p, pc):
                    pcol = jnp.full((16,), p, jnp.int32)
                    for cc in range(C // 16):
                        sl = pl.ds(cc * 16, 16)
                        v = jnp.maximum(
                            jnp.maximum(rbuf[p, sl], rbuf[_CHUNK + p, sl]),
                            rbuf[2 * _CHUNK + p, sl])
                        plsc.store_scatter(obuf, [rowidx[cc], pcol], v)
                    return pc
                lax.fori_loop(0, _CHUNK, pbody, 0, unroll=2)

                obase = pl.multiple_of(((g0 + c) * B + b) * C, 8)
                pltpu.sync_copy(obuf.at[:, pl.ds(0, _CHUNK)],
                                out_h.at[pl.ds(obase, C), :])
            return carry
        lax.fori_loop(0, ch // 2, outer, 0)

    return sc_kernel(table, idx3)


_GBLK = 8


def _stats_tc(inter, skipt, wpt, bpt, wst, bst, B, N, C):
    Co = wpt.shape[0]
    Cs = skipt.shape[1]
    NG = N // 128
    nb = NG // _GBLK

    def body(x_ref, s_ref, wpt_ref, bpt_ref, wst_ref, bst_ref, oi_ref, os_ref):
        first = pl.program_id(0) == 0
        x4 = x_ref[...].reshape(_GBLK, B, C, 128)
        s1i = jnp.zeros((Co, 1), jnp.float32)
        s2i = jnp.zeros((Co, 1), jnp.float32)
        for gi in range(_GBLK):
            for b in range(B):
                yit = jnp.dot(wpt_ref[...], x4[gi, b],
                              preferred_element_type=jnp.float32) + bpt_ref[...]
                s1i += jnp.sum(yit, axis=1, keepdims=True)
                s2i += jnp.sum(yit * yit, axis=1, keepdims=True)
        s1s = jnp.zeros((Cs, 1), jnp.float32)
        s2s = jnp.zeros((Cs, 1), jnp.float32)
        for b in range(B):
            yst = jnp.dot(wst_ref[...], s_ref[b],
                          preferred_element_type=jnp.float32) + bst_ref[...]
            s1s += jnp.sum(yst, axis=1, keepdims=True)
            s2s += jnp.sum(yst * yst, axis=1, keepdims=True)

        @pl.when(first)
        def _():
            oi_ref[...] = jnp.zeros_like(oi_ref)
            os_ref[...] = jnp.zeros_like(os_ref)

        oi_ref[...] += jnp.concatenate([s1i, s2i], axis=1)
        os_ref[...] += jnp.concatenate([s1s, s2s], axis=1)

    return pl.pallas_call(
        body,
        grid=(nb,),
        in_specs=[
            pl.BlockSpec((_GBLK * B * C, 128), lambda j: (j, 0)),
            pl.BlockSpec((B, Cs, _GBLK * 128), lambda j: (0, 0, j)),
            pl.BlockSpec(wpt.shape, lambda j: (0, 0)),
            pl.BlockSpec(bpt.shape, lambda j: (0, 0)),
            pl.BlockSpec(wst.shape, lambda j: (0, 0)),
            pl.BlockSpec(bst.shape, lambda j: (0, 0)),
        ],
        out_specs=[
            pl.BlockSpec((Co, 2), lambda j: (0, 0)),
            pl.BlockSpec((Cs, 2), lambda j: (0, 0)),
        ],
        out_shape=[
            jax.ShapeDtypeStruct((Co, 2), jnp.float32),
            jax.ShapeDtypeStruct((Cs, 2), jnp.float32),
        ],
    )(inter, skipt, wpt, bpt, wst, bst)


def _apply_tc(inter, skipt, coords3, sti, sts, wpt, bpt, wst, bst,
              gp, bep, gs, bes, B, N, C):
    Co = wpt.shape[0]
    Cs = skipt.shape[1]
    NG = N // 128
    nb = NG // _GBLK
    cnt = float(B * N)
    GB = _GBLK * B

    def body(sti_ref, sts_ref, x_ref, s_ref, c_ref, wpt_ref, bpt_ref,
             wst_ref, bst_ref, gp_ref, bep_ref, gs_ref, bes_ref, o_ref):
        sti_v = sti_ref[...]
        m_i = sti_v[:, 0:1] * (1.0 / cnt)
        v_i = sti_v[:, 1:2] * (1.0 / cnt) - m_i * m_i
        sc_i = gp_ref[...] * lax.rsqrt(v_i + _EPS)
        sh_i = bep_ref[...] - m_i * sc_i
        sts_v = sts_ref[...]
        m_s = sts_v[:, 0:1] * (1.0 / cnt)
        v_s = sts_v[:, 1:2] * (1.0 / cnt) - m_s * m_s
        sc_s = gs_ref[...] * lax.rsqrt(v_s + _EPS)
        sh_s = bes_ref[...] - m_s * sc_s

        o_ref[0:3, :, :] = c_ref[...]

        for b in range(B):
            yst = jnp.dot(wst_ref[...], s_ref[b],
                          preferred_element_type=jnp.float32) + bst_ref[...]
            zst = jnp.maximum(yst * sc_s + sh_s, 0.0)
            for gi in range(_GBLK):
                o_ref[3:3 + Cs, pl.ds(gi * B + b, 1), :] = (
                    zst[:, gi * 128:(gi + 1) * 128])[:, None, :]

        x4 = x_ref[...].reshape(_GBLK, B, C, 128)
        for gi in range(_GBLK):
            for b in range(B):
                yit = jnp.dot(wpt_ref[...], x4[gi, b],
                              preferred_element_type=jnp.float32) + bpt_ref[...]
                zit = jnp.maximum(yit * sc_i + sh_i, 0.0)
                o_ref[3 + Cs:3 + Cs + Co, pl.ds(gi * B + b, 1), :] = zit[:, None, :]

    return pl.pallas_call(
        body,
        grid=(nb,),
        in_specs=[
            pl.BlockSpec((Co, 2), lambda j: (0, 0)),
            pl.BlockSpec((Cs, 2), lambda j: (0, 0)),
            pl.BlockSpec((_GBLK * B * C, 128), lambda j: (j, 0)),
            pl.BlockSpec((B, Cs, _GBLK * 128), lambda j: (0, 0, j)),
            pl.BlockSpec((3, GB, 128), lambda j: (0, j, 0)),
            pl.BlockSpec(wpt.shape, lambda j: (0, 0)),
            pl.BlockSpec(bpt.shape, lambda j: (0, 0)),
            pl.BlockSpec(wst.shape, lambda j: (0, 0)),
            pl.BlockSpec(bst.shape, lambda j: (0, 0)),
            pl.BlockSpec(gp.shape, lambda j: (0, 0)),
            pl.BlockSpec(bep.shape, lambda j: (0, 0)),
            pl.BlockSpec(gs.shape, lambda j: (0, 0)),
            pl.BlockSpec(bes.shape, lambda j: (0, 0)),
        ],
        out_specs=pl.BlockSpec((3 + Cs + Co, GB, 128), lambda j: (0, j, 0)),
        out_shape=jax.ShapeDtypeStruct((3 + Cs + Co, NG * B, 128), jnp.float32),
    )(sti, sts, inter, skipt, coords3, wpt, bpt, wst, bst, gp, bep, gs, bes)


def kernel(curr_coords, curr_feats, skip_coords, skip_feats, upsampling_idxs,
           W_proj, b_proj, g_proj, be_proj, W_skip, b_skip, g_skip, be_skip):
    B, M, C = curr_feats.shape
    _, N, K = upsampling_idxs.shape
    Co = W_proj.shape[1]

    idx3 = (upsampling_idxs.reshape(B, N // 128, 128, K)
            .transpose(3, 1, 0, 2).reshape(K, N // 128, B * 128))
    table = curr_feats.reshape(B * M, C)
    skipt = jnp.transpose(skip_feats, (0, 2, 1))
    coords3 = (skip_coords.reshape(B, N // 128, 128, 3)
               .transpose(3, 1, 0, 2).reshape(3, (N // 128) * B, 128))

    inter = _gather_max_sc(table, idx3, B, M, N, K, C)

    wpt = W_proj.T
    wst = W_skip.T
    col = lambda a: a.reshape(-1, 1)
    sti, sts = _stats_tc(inter, skipt, wpt, col(b_proj), wst, col(b_skip),
                         B, N, C)
    out3 = _apply_tc(inter, skipt, coords3, sti, sts,
                     wpt, col(b_proj), wst, col(b_skip),
                     col(g_proj), col(be_proj), col(g_skip), col(be_skip),
                     B, N, C)
    return (out3.reshape(3 + 2 * Co, N // 128, B, 128)
            .transpose(2, 1, 3, 0).reshape(B, N, 3 + 2 * Co))
